# Initial kernel scaffold; baseline (speedup 1.0000x reference)
import jax
import jax.numpy as jnp
from jax.experimental import pallas as pl


def kernel(a_hat, x, w1, b1, w2, b2, wp, bp):
    raise NotImplementedError("write your pallas kernel here")


# speedup vs baseline: 1.0387x; 1.0387x over previous
import functools

import jax
import jax.numpy as jnp
from jax.experimental import pallas as pl
from jax.experimental.pallas import tpu as pltpu

_SLOPE = 0.01
_LANE = 128


def _lrelu(v):
    return jnp.where(v >= 0, v, _SLOPE * v)


def _ceil_to(v, m):
    return (v + m - 1) // m * m


def _pad_to(a, rows, cols, dtype=None):
    r, c = a.shape
    out = jnp.pad(a, ((0, rows - r), (0, cols - c)))
    return out.astype(dtype) if dtype is not None else out


def _resident(shape, index_map):
    try:
        return pl.BlockSpec(shape, index_map, pipeline_mode=pl.Buffered(1))
    except TypeError:
        return pl.BlockSpec(shape, index_map)


def _layer1_body(a_ref, x_ref, w1_ref, w2t_ref, w2b_ref, b1_ref, yw_ref,
                 acc_ref, *, tile):
    k = pl.program_id(1)

    @pl.when(k == 0)
    def _():
        acc_ref[...] = jnp.zeros_like(acc_ref)

    a = a_ref[...].astype(jnp.bfloat16)
    koff = pl.multiple_of(k * tile, tile)
    src = x_ref[pl.ds(koff, tile), :].astype(jnp.bfloat16)
    acc_ref[...] += jnp.dot(a, src, preferred_element_type=jnp.float32)

    @pl.when(k == pl.num_programs(1) - 1)
    def _():
        ax = acc_ref[...].astype(jnp.bfloat16)
        y1 = _lrelu(jnp.dot(ax, w1_ref[...],
                            preferred_element_type=jnp.float32) + b1_ref[...])
        i = pl.program_id(0)
        ioff = pl.multiple_of(i * tile, tile)
        xl = _lrelu(x_ref[pl.ds(ioff, tile), :]).astype(jnp.bfloat16)
        yw = (jnp.dot(xl, w2t_ref[...], preferred_element_type=jnp.float32)
              + jnp.dot(y1.astype(jnp.bfloat16), w2b_ref[...],
                        preferred_element_type=jnp.float32))
        yw_ref[...] = yw.astype(yw_ref.dtype)


def _layer2_body(a_ref, yw_ref, b2_ref, wp_ref, bp_ref, z_ref, out_ref,
                 acc_ref, *, tile):
    k = pl.program_id(1)

    @pl.when(k == 0)
    def _():
        acc_ref[...] = jnp.zeros_like(acc_ref)

    a = a_ref[...].astype(jnp.bfloat16)
    koff = pl.multiple_of(k * tile, tile)
    acc_ref[...] += jnp.dot(a, yw_ref[pl.ds(koff, tile), :],
                            preferred_element_type=jnp.float32)

    @pl.when(k == pl.num_programs(1) - 1)
    def _():
        z = _lrelu(acc_ref[...] + b2_ref[...])
        z_ref[...] = z
        out_ref[...] = (jnp.dot(z.astype(jnp.bfloat16), wp_ref[...],
                                preferred_element_type=jnp.float32)
                        + bp_ref[...])


@jax.jit
def _forward(a_hat, x, w1, b1, w2, b2, wp, bp):
    n, d = x.shape
    tile = 256
    n_pad = _ceil_to(n, tile)
    d_pad = _ceil_to(d, _LANE)
    p_pad = _LANE

    a_p = _pad_to(a_hat, n_pad, n_pad)
    x_p = _pad_to(x, n_pad, d_pad)
    w1_p = _pad_to(w1, d_pad, d_pad, jnp.bfloat16)
    w2t_p = _pad_to(w2[:d], d_pad, d_pad, jnp.bfloat16)
    w2b_p = _pad_to(w2[d:], d_pad, d_pad, jnp.bfloat16)
    wp_p = _pad_to(wp, d_pad, p_pad, jnp.bfloat16)
    b1_p = _pad_to(b1, 1, d_pad)
    b2_p = _pad_to(b2, 1, d_pad)
    bp_p = _pad_to(bp, 1, p_pad)

    n_blk = n_pad // tile
    grid = (n_blk, n_blk)
    cparams = pltpu.CompilerParams(
        dimension_semantics=("parallel", "arbitrary"),
        vmem_limit_bytes=48 * 1024 * 1024)
    inv = lambda shape: _resident(shape, lambda i, k: (0, 0))

    flops1 = 2 * n_pad * n_pad * d_pad + 4 * n_pad * d_pad * d_pad
    bytes1 = 4 * (a_p.size + x_p.size) + 6 * d_pad * d_pad + 2 * n_pad * d_pad
    yw = pl.pallas_call(
        functools.partial(_layer1_body, tile=tile),
        out_shape=jax.ShapeDtypeStruct((n_pad, d_pad), jnp.bfloat16),
        grid=grid,
        in_specs=[
            pl.BlockSpec((tile, tile), lambda i, k: (i, k)),
            inv((n_pad, d_pad)),
            inv((d_pad, d_pad)),
            inv((d_pad, d_pad)),
            inv((d_pad, d_pad)),
            inv((1, d_pad)),
        ],
        out_specs=pl.BlockSpec((tile, d_pad), lambda i, k: (i, 0)),
        scratch_shapes=[pltpu.VMEM((tile, d_pad), jnp.float32)],
        compiler_params=cparams,
        cost_estimate=pl.CostEstimate(flops=int(flops1), transcendentals=0,
                                      bytes_accessed=int(bytes1)),
    )(a_p, x_p, w1_p, w2t_p, w2b_p, b1_p)

    flops2 = 2 * n_pad * n_pad * d_pad + 2 * n_pad * d_pad * p_pad
    bytes2 = (4 * a_p.size + 2 * n_pad * d_pad + 2 * d_pad * p_pad
              + 4 * n_pad * (d_pad + p_pad))
    z_pad, out_pad = pl.pallas_call(
        functools.partial(_layer2_body, tile=tile),
        out_shape=(jax.ShapeDtypeStruct((n_pad, d_pad), jnp.float32),
                   jax.ShapeDtypeStruct((n_pad, p_pad), jnp.float32)),
        grid=grid,
        in_specs=[
            pl.BlockSpec((tile, tile), lambda i, k: (i, k)),
            inv((n_pad, d_pad)),
            inv((1, d_pad)),
            inv((d_pad, p_pad)),
            inv((1, p_pad)),
        ],
        out_specs=(pl.BlockSpec((tile, d_pad), lambda i, k: (i, 0)),
                   pl.BlockSpec((tile, p_pad), lambda i, k: (i, 0))),
        scratch_shapes=[pltpu.VMEM((tile, d_pad), jnp.float32)],
        compiler_params=cparams,
        cost_estimate=pl.CostEstimate(flops=int(flops2), transcendentals=0,
                                      bytes_accessed=int(bytes2)),
    )(a_p, yw, b2_p, wp_p, bp_p)

    z = z_pad[:n, :d]
    out = out_pad[:n, :3]
    return z, out[:, 0], out[:, 1], out[:, 2]


def kernel(a_hat, x, w1, b1, w2, b2, wp, bp):
    return _forward(a_hat, x, w1, b1, w2, b2, wp, bp)


# speedup vs baseline: 4.9517x; 4.7671x over previous
import functools

import jax
import jax.numpy as jnp
from jax.experimental import pallas as pl
from jax.experimental.pallas import tpu as pltpu

_SLOPE = 0.01
_LANE = 128


def _lrelu(v):
    return jnp.where(v >= 0, v, _SLOPE * v)


def _ceil_to(v, m):
    return (v + m - 1) // m * m


def _pad_to(a, rows, cols, dtype=None):
    r, c = a.shape
    out = jnp.pad(a, ((0, rows - r), (0, cols - c)))
    return out.astype(dtype) if dtype is not None else out


def _resident(shape, index_map):
    try:
        return pl.BlockSpec(shape, index_map, pipeline_mode=pl.Buffered(1))
    except TypeError:
        return pl.BlockSpec(shape, index_map)


def _gnn_body(a_ref, x_ref, w1_ref, w2t_ref, w2b_ref, b1_ref, b2_ref,
              wp_ref, bp_ref, z_ref, out_ref, yw_ref, *, tile):
    s = pl.program_id(0)
    i = pl.program_id(1)
    a = a_ref[...].astype(jnp.bfloat16)
    ioff = pl.multiple_of(i * tile, tile)

    @pl.when(s == 0)
    def _():
        t = jnp.dot(a, x_ref[...], preferred_element_type=jnp.float32)
        y1 = _lrelu(jnp.dot(t.astype(jnp.bfloat16), w1_ref[...],
                            preferred_element_type=jnp.float32) + b1_ref[...])
        xl = _lrelu(x_ref[pl.ds(ioff, tile), :].astype(jnp.float32))
        yw = (jnp.dot(xl.astype(jnp.bfloat16), w2t_ref[...],
                      preferred_element_type=jnp.float32)
              + jnp.dot(y1.astype(jnp.bfloat16), w2b_ref[...],
                        preferred_element_type=jnp.float32))
        yw_ref[pl.ds(ioff, tile), :] = yw.astype(jnp.bfloat16)

    @pl.when(s == 1)
    def _():
        t = jnp.dot(a, yw_ref[...], preferred_element_type=jnp.float32)
        z = _lrelu(t + b2_ref[...])
        z_ref[...] = z
        out_ref[...] = (jnp.dot(z.astype(jnp.bfloat16), wp_ref[...],
                                preferred_element_type=jnp.float32)
                        + bp_ref[...])


@jax.jit
def _forward(a_hat, x, w1, b1, w2, b2, wp, bp):
    n, d = x.shape
    tile = 256
    n_pad = _ceil_to(n, tile)
    d_pad = _ceil_to(d, _LANE)
    p_pad = _LANE

    a_p = _pad_to(a_hat, n_pad, n_pad)
    x_p = _pad_to(x, n_pad, d_pad, jnp.bfloat16)
    w1_p = _pad_to(w1, d_pad, d_pad, jnp.bfloat16)
    w2t_p = _pad_to(w2[:d], d_pad, d_pad, jnp.bfloat16)
    w2b_p = _pad_to(w2[d:], d_pad, d_pad, jnp.bfloat16)
    wp_p = _pad_to(wp, d_pad, p_pad, jnp.bfloat16)
    b1_p = _pad_to(b1, 1, d_pad)
    b2_p = _pad_to(b2, 1, d_pad)
    bp_p = _pad_to(bp, 1, p_pad)

    n_blk = n_pad // tile
    inv = lambda shape: _resident(shape, lambda s, i: (0, 0))

    flops = 4 * n_pad * n_pad * d_pad + 6 * n_pad * d_pad * d_pad
    nbytes = 8 * a_p.size + 2 * x_p.size + 6 * d_pad * d_pad \
        + 4 * n_pad * (d_pad + p_pad)
    out_idx = lambda s, i: (jnp.where(s == 0, n_blk, i), 0)
    z_pad, out_pad = pl.pallas_call(
        functools.partial(_gnn_body, tile=tile),
        out_shape=(jax.ShapeDtypeStruct((n_pad + tile, d_pad), jnp.float32),
                   jax.ShapeDtypeStruct((n_pad + tile, p_pad), jnp.float32)),
        grid=(2, n_blk),
        in_specs=[
            pl.BlockSpec((tile, n_pad), lambda s, i: (i, 0)),
            inv((n_pad, d_pad)),
            inv((d_pad, d_pad)),
            inv((d_pad, d_pad)),
            inv((d_pad, d_pad)),
            inv((1, d_pad)),
            inv((1, d_pad)),
            inv((d_pad, p_pad)),
            inv((1, p_pad)),
        ],
        out_specs=(pl.BlockSpec((tile, d_pad), out_idx),
                   pl.BlockSpec((tile, p_pad), out_idx)),
        scratch_shapes=[pltpu.VMEM((n_pad, d_pad), jnp.bfloat16)],
        compiler_params=pltpu.CompilerParams(
            dimension_semantics=("arbitrary", "arbitrary"),
            vmem_limit_bytes=48 * 1024 * 1024),
        cost_estimate=pl.CostEstimate(flops=int(flops), transcendentals=0,
                                      bytes_accessed=int(nbytes)),
    )(a_p, x_p, w1_p, w2t_p, w2b_p, b1_p, b2_p, wp_p, bp_p)

    z = z_pad[:n, :d]
    out = out_pad[:n, :3]
    return z, out[:, 0], out[:, 1], out[:, 2]


def kernel(a_hat, x, w1, b1, w2, b2, wp, bp):
    return _forward(a_hat, x, w1, b1, w2, b2, wp, bp)


# speedup vs baseline: 5.4001x; 1.0905x over previous
import functools

import jax
import jax.numpy as jnp
from jax.experimental import pallas as pl
from jax.experimental.pallas import tpu as pltpu

_SLOPE = 0.01
_LANE = 128


def _lrelu(v):
    return jnp.where(v >= 0, v, _SLOPE * v)


def _ceil_to(v, m):
    return (v + m - 1) // m * m


def _pad_to(a, rows, cols, dtype=None):
    r, c = a.shape
    out = jnp.pad(a, ((0, rows - r), (0, cols - c)))
    return out.astype(dtype) if dtype is not None else out


def _resident(shape, index_map):
    try:
        return pl.BlockSpec(shape, index_map, pipeline_mode=pl.Buffered(1))
    except TypeError:
        return pl.BlockSpec(shape, index_map)


def _gnn_body(a_ref, x_ref, w1_ref, w2t_ref, w2b_ref, b1_ref, b2_ref,
              wp_ref, bp_ref, z_ref, out_ref, abf_ref, yw_ref, *, tile):
    s = pl.program_id(0)
    i = pl.program_id(1)
    ioff = pl.multiple_of(i * tile, tile)

    @pl.when(s == 0)
    def _():
        abf_ref[pl.ds(ioff, tile), :] = a_ref[...].astype(jnp.bfloat16)
        a = abf_ref[pl.ds(ioff, tile), :]
        t = jnp.dot(a, x_ref[...], preferred_element_type=jnp.float32)
        y1 = _lrelu(jnp.dot(t.astype(jnp.bfloat16), w1_ref[...],
                            preferred_element_type=jnp.float32) + b1_ref[...])
        xl = _lrelu(x_ref[pl.ds(ioff, tile), :].astype(jnp.float32))
        yw = (jnp.dot(xl.astype(jnp.bfloat16), w2t_ref[...],
                      preferred_element_type=jnp.float32)
              + jnp.dot(y1.astype(jnp.bfloat16), w2b_ref[...],
                        preferred_element_type=jnp.float32))
        yw_ref[pl.ds(ioff, tile), :] = yw.astype(jnp.bfloat16)

    @pl.when(s == 1)
    def _():
        a = abf_ref[pl.ds(ioff, tile), :]
        t = jnp.dot(a, yw_ref[...], preferred_element_type=jnp.float32)
        z = _lrelu(t + b2_ref[...])
        z_ref[...] = z
        out_ref[...] = (jnp.dot(z.astype(jnp.bfloat16), wp_ref[...],
                                preferred_element_type=jnp.float32)
                        + bp_ref[...])


@jax.jit
def _forward(a_hat, x, w1, b1, w2, b2, wp, bp):
    n, d = x.shape
    tile = 256
    n_pad = _ceil_to(n, tile)
    d_pad = _ceil_to(d, _LANE)
    p_pad = _LANE

    a_p = _pad_to(a_hat, n_pad, n_pad)
    x_p = _pad_to(x, n_pad, d_pad, jnp.bfloat16)
    w1_p = _pad_to(w1, d_pad, d_pad, jnp.bfloat16)
    w2t_p = _pad_to(w2[:d], d_pad, d_pad, jnp.bfloat16)
    w2b_p = _pad_to(w2[d:], d_pad, d_pad, jnp.bfloat16)
    wp_p = _pad_to(wp, d_pad, p_pad, jnp.bfloat16)
    b1_p = _pad_to(b1, 1, d_pad)
    b2_p = _pad_to(b2, 1, d_pad)
    bp_p = _pad_to(bp, 1, p_pad)

    n_blk = n_pad // tile
    inv = lambda shape: _resident(shape, lambda s, i: (0, 0))

    a_idx = lambda s, i: (jnp.where(s == 0, i, n_blk - 1), 0)
    out_idx = lambda s, i: (jnp.where(s == 0, n_blk, i), 0)

    flops = 4 * n_pad * n_pad * d_pad + 6 * n_pad * d_pad * d_pad
    nbytes = 4 * a_p.size + 2 * x_p.size + 6 * d_pad * d_pad \
        + 4 * n_pad * (d_pad + p_pad)
    z_pad, out_pad = pl.pallas_call(
        functools.partial(_gnn_body, tile=tile),
        out_shape=(jax.ShapeDtypeStruct((n_pad + tile, d_pad), jnp.float32),
                   jax.ShapeDtypeStruct((n_pad + tile, p_pad), jnp.float32)),
        grid=(2, n_blk),
        in_specs=[
            pl.BlockSpec((tile, n_pad), a_idx),
            inv((n_pad, d_pad)),
            inv((d_pad, d_pad)),
            inv((d_pad, d_pad)),
            inv((d_pad, d_pad)),
            inv((1, d_pad)),
            inv((1, d_pad)),
            inv((d_pad, p_pad)),
            inv((1, p_pad)),
        ],
        out_specs=(pl.BlockSpec((tile, d_pad), out_idx),
                   pl.BlockSpec((tile, p_pad), out_idx)),
        scratch_shapes=[pltpu.VMEM((n_pad, n_pad), jnp.bfloat16),
                        pltpu.VMEM((n_pad, d_pad), jnp.bfloat16)],
        compiler_params=pltpu.CompilerParams(
            dimension_semantics=("arbitrary", "arbitrary"),
            vmem_limit_bytes=57 * 1024 * 1024),
        cost_estimate=pl.CostEstimate(flops=int(flops), transcendentals=0,
                                      bytes_accessed=int(nbytes)),
    )(a_p, x_p, w1_p, w2t_p, w2b_p, b1_p, b2_p, wp_p, bp_p)

    z = z_pad[:n, :d]
    out = out_pad[:n, :3]
    return z, out[:, 0], out[:, 1], out[:, 2]


def kernel(a_hat, x, w1, b1, w2, b2, wp, bp):
    return _forward(a_hat, x, w1, b1, w2, b2, wp, bp)


# speedup vs baseline: 6.1868x; 1.1457x over previous
import functools

import jax
import jax.numpy as jnp
from jax.experimental import pallas as pl
from jax.experimental.pallas import tpu as pltpu

_SLOPE = 0.01
_LANE = 128


def _lrelu(v):
    return jnp.where(v >= 0, v, _SLOPE * v)


def _ceil_to(v, m):
    return (v + m - 1) // m * m


def _pad_to(a, rows, cols, dtype=None):
    r, c = a.shape
    out = jnp.pad(a, ((0, rows - r), (0, cols - c)))
    return out.astype(dtype) if dtype is not None else out


def _resident(shape, index_map):
    try:
        return pl.BlockSpec(shape, index_map, pipeline_mode=pl.Buffered(1))
    except TypeError:
        return pl.BlockSpec(shape, index_map)


def _gnn_body(a_ref, x_ref, w1_ref, w2t_ref, w2b_ref, b1_ref, b2_ref,
              wp_ref, bp_ref, z_ref, out_ref, abf_ref, yw_ref, *, tile):
    s = pl.program_id(0)
    i = pl.program_id(1)
    ioff = pl.multiple_of(i * tile, tile)

    @pl.when(s == 0)
    def _():
        abf_ref[pl.ds(ioff, tile), :] = a_ref[...].astype(jnp.bfloat16)
        a = abf_ref[pl.ds(ioff, tile), :]
        t = jnp.dot(a, x_ref[...], preferred_element_type=jnp.float32)
        y1 = _lrelu(jnp.dot(t.astype(jnp.bfloat16), w1_ref[...],
                            preferred_element_type=jnp.float32) + b1_ref[...])
        xl = _lrelu(x_ref[pl.ds(ioff, tile), :].astype(jnp.float32))
        yw = (jnp.dot(xl.astype(jnp.bfloat16), w2t_ref[...],
                      preferred_element_type=jnp.float32)
              + jnp.dot(y1.astype(jnp.bfloat16), w2b_ref[...],
                        preferred_element_type=jnp.float32))
        yw_ref[pl.ds(ioff, tile), :] = yw.astype(jnp.bfloat16)

    @pl.when(s == 1)
    def _():
        a = abf_ref[pl.ds(ioff, tile), :]
        t = jnp.dot(a, yw_ref[...], preferred_element_type=jnp.float32)
        z = _lrelu(t + b2_ref[...])
        z_ref[...] = z
        out_ref[...] = (jnp.dot(z.astype(jnp.bfloat16), wp_ref[...],
                                preferred_element_type=jnp.float32)
                        + bp_ref[...])


@jax.jit
def _forward(a_hat, x, w1, b1, w2, b2, wp, bp):
    n, d = x.shape
    tile = 512
    n_pad = _ceil_to(n, tile)
    d_pad = _ceil_to(d, _LANE)
    p_pad = _LANE

    a_p = _pad_to(a_hat, n_pad, n_pad)
    x_p = _pad_to(x, n_pad, d_pad, jnp.bfloat16)
    w1_p = _pad_to(w1, d_pad, d_pad, jnp.bfloat16)
    w2t_p = _pad_to(w2[:d], d_pad, d_pad, jnp.bfloat16)
    w2b_p = _pad_to(w2[d:], d_pad, d_pad, jnp.bfloat16)
    wp_p = _pad_to(wp, d_pad, p_pad, jnp.bfloat16)
    b1_p = _pad_to(b1, 1, d_pad)
    b2_p = _pad_to(b2, 1, d_pad)
    bp_p = _pad_to(bp, 1, p_pad)

    n_blk = n_pad // tile
    inv = lambda shape: _resident(shape, lambda s, i: (0, 0))

    a_idx = lambda s, i: (jnp.where(s == 0, i, n_blk - 1), 0)
    out_idx = lambda s, i: (jnp.where(s == 0, n_blk, i), 0)

    flops = 4 * n_pad * n_pad * d_pad + 6 * n_pad * d_pad * d_pad
    nbytes = 4 * a_p.size + 2 * x_p.size + 6 * d_pad * d_pad \
        + 4 * n_pad * (d_pad + p_pad)
    z_pad, out_pad = pl.pallas_call(
        functools.partial(_gnn_body, tile=tile),
        out_shape=(jax.ShapeDtypeStruct((n_pad + tile, d_pad), jnp.float32),
                   jax.ShapeDtypeStruct((n_pad + tile, p_pad), jnp.float32)),
        grid=(2, n_blk),
        in_specs=[
            pl.BlockSpec((tile, n_pad), a_idx),
            inv((n_pad, d_pad)),
            inv((d_pad, d_pad)),
            inv((d_pad, d_pad)),
            inv((d_pad, d_pad)),
            inv((1, d_pad)),
            inv((1, d_pad)),
            inv((d_pad, p_pad)),
            inv((1, p_pad)),
        ],
        out_specs=(pl.BlockSpec((tile, d_pad), out_idx),
                   pl.BlockSpec((tile, p_pad), out_idx)),
        scratch_shapes=[pltpu.VMEM((n_pad, n_pad), jnp.bfloat16),
                        pltpu.VMEM((n_pad, d_pad), jnp.bfloat16)],
        compiler_params=pltpu.CompilerParams(
            dimension_semantics=("arbitrary", "arbitrary"),
            vmem_limit_bytes=57 * 1024 * 1024),
        cost_estimate=pl.CostEstimate(flops=int(flops), transcendentals=0,
                                      bytes_accessed=int(nbytes)),
    )(a_p, x_p, w1_p, w2t_p, w2b_p, b1_p, b2_p, wp_p, bp_p)

    z = z_pad[:n, :d]
    out = out_pad[:n, :3]
    return z, out[:, 0], out[:, 1], out[:, 2]


def kernel(a_hat, x, w1, b1, w2, b2, wp, bp):
    return _forward(a_hat, x, w1, b1, w2, b2, wp, bp)


# speedup vs baseline: 6.2304x; 1.0070x over previous
import functools

import jax
import jax.numpy as jnp
from jax.experimental import pallas as pl
from jax.experimental.pallas import tpu as pltpu

_SLOPE = 0.01
_LANE = 128


def _lrelu(v):
    return jnp.where(v >= 0, v, _SLOPE * v)


def _ceil_to(v, m):
    return (v + m - 1) // m * m


def _pad_to(a, rows, cols, dtype=None):
    r, c = a.shape
    out = a if (r, c) == (rows, cols) else jnp.pad(
        a, ((0, rows - r), (0, cols - c)))
    return out.astype(dtype) if dtype is not None else out


def _resident(shape, index_map):
    try:
        return pl.BlockSpec(shape, index_map, pipeline_mode=pl.Buffered(1))
    except TypeError:
        return pl.BlockSpec(shape, index_map)


def _gnn_body(a_ref, x_ref, w1_ref, w2t_ref, w2b_ref, b1_ref, b2_ref,
              wp_ref, bp_ref, z_ref, out_ref, abf_ref, yw_ref, *, tile):
    s = pl.program_id(0)
    i = pl.program_id(1)
    ioff = pl.multiple_of(i * tile, tile)

    @pl.when(s == 0)
    def _():
        abf_ref[pl.ds(ioff, tile), :] = a_ref[...].astype(jnp.bfloat16)
        a = abf_ref[pl.ds(ioff, tile), :]
        t = jnp.dot(a, x_ref[...], preferred_element_type=jnp.float32)
        y1 = _lrelu(jnp.dot(t.astype(jnp.bfloat16), w1_ref[...],
                            preferred_element_type=jnp.float32) + b1_ref[...])
        xl = _lrelu(x_ref[pl.ds(ioff, tile), :].astype(jnp.float32))
        yw = (jnp.dot(xl.astype(jnp.bfloat16), w2t_ref[...],
                      preferred_element_type=jnp.float32)
              + jnp.dot(y1.astype(jnp.bfloat16), w2b_ref[...],
                        preferred_element_type=jnp.float32))
        yw_ref[pl.ds(ioff, tile), :] = yw.astype(jnp.bfloat16)

    @pl.when(s == 1)
    def _():
        a = abf_ref[pl.ds(ioff, tile), :]
        t = jnp.dot(a, yw_ref[...], preferred_element_type=jnp.float32)
        z = _lrelu(t + b2_ref[...])
        z_ref[...] = z
        out_ref[...] = (jnp.dot(z.astype(jnp.bfloat16), wp_ref[...],
                                preferred_element_type=jnp.float32)
                        + bp_ref[...])


@jax.jit
def _forward(a_hat, x, w1, b1, w2, b2, wp, bp):
    n, d = x.shape
    tile = 512
    n_pad = _ceil_to(n, tile)
    d_pad = _ceil_to(d, _LANE)
    p_pad = _LANE

    a_p = _pad_to(a_hat, n_pad, n_pad)
    x_p = _pad_to(x, n_pad, d_pad, jnp.bfloat16)
    w1_p = _pad_to(w1, d_pad, d_pad, jnp.bfloat16)
    w2t_p = _pad_to(w2[:d], d_pad, d_pad, jnp.bfloat16)
    w2b_p = _pad_to(w2[d:], d_pad, d_pad, jnp.bfloat16)
    wp_p = _pad_to(wp, d_pad, p_pad, jnp.bfloat16)
    b1_p = _pad_to(b1, 1, d_pad)
    b2_p = _pad_to(b2, 1, d_pad)
    bp_p = _pad_to(bp, 1, p_pad)

    n_blk = n_pad // tile
    inv = lambda shape: _resident(shape, lambda s, i: (0, 0))

    a_idx = lambda s, i: (jnp.where(s == 0, i, n_blk - 1), 0)
    out_idx = lambda s, i: (jnp.where(s == 0, 0, i), 0)

    flops = 4 * n_pad * n_pad * d_pad + 6 * n_pad * d_pad * d_pad
    nbytes = 4 * a_p.size + 2 * x_p.size + 6 * d_pad * d_pad \
        + 4 * n_pad * (d_pad + p_pad)
    z_pad, out_pad = pl.pallas_call(
        functools.partial(_gnn_body, tile=tile),
        out_shape=(jax.ShapeDtypeStruct((n_pad, d_pad), jnp.float32),
                   jax.ShapeDtypeStruct((n_pad, p_pad), jnp.float32)),
        grid=(2, n_blk),
        in_specs=[
            pl.BlockSpec((tile, n_pad), a_idx),
            inv((n_pad, d_pad)),
            inv((d_pad, d_pad)),
            inv((d_pad, d_pad)),
            inv((d_pad, d_pad)),
            inv((1, d_pad)),
            inv((1, d_pad)),
            inv((d_pad, p_pad)),
            inv((1, p_pad)),
        ],
        out_specs=(pl.BlockSpec((tile, d_pad), out_idx),
                   pl.BlockSpec((tile, p_pad), out_idx)),
        scratch_shapes=[pltpu.VMEM((n_pad, n_pad), jnp.bfloat16),
                        pltpu.VMEM((n_pad, d_pad), jnp.bfloat16)],
        compiler_params=pltpu.CompilerParams(
            dimension_semantics=("arbitrary", "arbitrary"),
            vmem_limit_bytes=57 * 1024 * 1024),
        cost_estimate=pl.CostEstimate(flops=int(flops), transcendentals=0,
                                      bytes_accessed=int(nbytes)),
    )(a_p, x_p, w1_p, w2t_p, w2b_p, b1_p, b2_p, wp_p, bp_p)

    z = z_pad if (n, d) == (n_pad, d_pad) else z_pad[:n, :d]
    out = out_pad[:n, :3]
    return z, out[:, 0], out[:, 1], out[:, 2]


def kernel(a_hat, x, w1, b1, w2, b2, wp, bp):
    return _forward(a_hat, x, w1, b1, w2, b2, wp, bp)
